# Initial kernel scaffold; baseline (speedup 1.0000x reference)
import jax
import jax.numpy as jnp
from jax.experimental import pallas as pl


def kernel(x, weight, bias):
    raise NotImplementedError("write your pallas kernel here")


# speedup vs baseline: 1.0165x; 1.0165x over previous
import functools

import jax
import jax.numpy as jnp
from jax import lax
from jax.experimental import pallas as pl
from jax.experimental.pallas import tpu as pltpu


def _pos_head_kernel(x_ref, w_ref, b_ref, o_ref, *, H, W, naxis):
    tn = o_ref.shape[2]
    off = (jnp.dot(w_ref[...], x_ref[0], preferred_element_type=jnp.float32)
           + b_ref[...])
    nio = lax.broadcasted_iota(jnp.int32, (1, tn), 1)
    ch = lax.broadcasted_iota(jnp.int32, (off.shape[0], 1), 0)
    is_x = ch < naxis
    base = jnp.where(is_x, nio % W, nio // W).astype(jnp.float32)
    bound = jnp.where(is_x, float(W - 1), float(H - 1))
    o_ref[0] = jnp.clip(off + base, 0.0, bound)


def _position_head(x2, wp, bp, *, H, W):
    B, C, N = x2.shape
    Cout = wp.shape[0]
    kern = functools.partial(_pos_head_kernel, H=H, W=W, naxis=Cout // 2)
    return pl.pallas_call(
        kern,
        out_shape=jax.ShapeDtypeStruct((B, Cout, N), jnp.float32),
        grid=(B,),
        in_specs=[
            pl.BlockSpec((1, C, N), lambda b: (b, 0, 0)),
            pl.BlockSpec((Cout, C), lambda b: (0, 0)),
            pl.BlockSpec((Cout, 1), lambda b: (0, 0)),
        ],
        out_specs=pl.BlockSpec((1, Cout, N), lambda b: (b, 0, 0)),
        compiler_params=pltpu.CompilerParams(
            dimension_semantics=("parallel",)),
    )(x2, wp, bp)


def _sample_kernel(pxy_ref, x_ref, o_ref, *, H, W):
    tS = o_ref.shape[2]
    px = pxy_ref[0, 0:1, :]
    py = pxy_ref[0, 1:2, :]

    x0f = jnp.floor(px)
    y0f = jnp.floor(py)
    wx = px - x0f
    wy = py - y0f
    x0 = x0f.astype(jnp.int32)
    y0 = y0f.astype(jnp.int32)
    x1 = jnp.minimum(x0 + 1, W - 1)
    y1 = jnp.minimum(y0 + 1, H - 1)

    wio = lax.broadcasted_iota(jnp.int32, (W, tS), 0)
    colT = (jnp.where(wio == x0, 1.0 - wx, 0.0)
            + jnp.where(wio == x1, wx, 0.0)).astype(jnp.bfloat16)
    hio = lax.broadcasted_iota(jnp.int32, (H, tS), 0)
    rowT = (jnp.where(hio == y0, 1.0 - wy, 0.0)
            + jnp.where(hio == y1, wy, 0.0)).astype(jnp.bfloat16)

    m = (rowT[:, None, :] * colT[None, :, :]).reshape(H * W, tS)
    xb = x_ref[0].astype(jnp.bfloat16)
    o_ref[0] = jnp.dot(xb, m, preferred_element_type=jnp.float32)


def _bilinear_sample(pxy, x_pix, *, H, W, tS):
    BG, Cg, N = x_pix.shape
    S = pxy.shape[2]
    kern = functools.partial(_sample_kernel, H=H, W=W)
    return pl.pallas_call(
        kern,
        out_shape=jax.ShapeDtypeStruct((BG, Cg, S), jnp.float32),
        grid=(BG, S // tS),
        in_specs=[
            pl.BlockSpec((1, 2, tS), lambda i, j: (i, 0, j)),
            pl.BlockSpec((1, Cg, N), lambda i, j: (i, 0, 0)),
        ],
        out_specs=pl.BlockSpec((1, Cg, tS), lambda i, j: (i, 0, j)),
        compiler_params=pltpu.CompilerParams(
            dimension_semantics=("parallel", "parallel")),
    )(pxy, x_pix)


def _init_pos(scale, groups):
    h = (jnp.arange(scale, dtype=jnp.float32) - (scale - 1) / 2.0) / scale
    t0 = jnp.broadcast_to(h[None, :], (scale, scale))
    t1 = jnp.broadcast_to(h[:, None], (scale, scale))
    return jnp.tile(jnp.stack([t0, t1]), (1, groups, 1)).reshape(-1)


def kernel(x, weight, bias):
    B, C, H, W = x.shape
    Cout = weight.shape[0]
    G, s = 4, 2
    Cg = C // G
    S = s * H * W * s
    N = H * W

    wp = (weight * 0.25).astype(jnp.float32)
    bp = (bias * 0.25 + _init_pos(s, G)).reshape(Cout, 1).astype(jnp.float32)

    pos = _position_head(x.reshape(B, C, N), wp, bp, H=H, W=W)

    pxy = (pos.reshape(B, 2, G, s, s, H, W)
           .transpose(0, 2, 1, 5, 3, 6, 4)
           .reshape(B * G, 2, S))

    x_pix = x.reshape(B * G, Cg, N)

    tS = 2048 if S % 2048 == 0 else S
    samp = _bilinear_sample(pxy, x_pix, H=H, W=W, tS=tS)
    return samp.reshape(B, C, s * H, s * W)


# speedup vs baseline: 3.2681x; 3.2150x over previous
import functools

import jax
import jax.numpy as jnp
from jax import lax
from jax.experimental import pallas as pl
from jax.experimental.pallas import tpu as pltpu


def _pos_head_kernel(x_ref, w_ref, b_ref, o_ref, *, H, W, s):
    N = x_ref.shape[2]
    Cout = w_ref.shape[0]
    sWs = s * W * s
    off = (jnp.dot(w_ref[...], x_ref[0], preferred_element_type=jnp.float32)
           + b_ref[...])
    nio = lax.broadcasted_iota(jnp.int32, (1, N), 1)
    ch = lax.broadcasted_iota(jnp.int32, (Cout, 1), 0)
    is_x = (ch % 2) == 0
    base = jnp.where(is_x, nio % W, nio // W).astype(jnp.float32)
    bound = jnp.where(is_x, float(W - 1), float(H - 1))
    pos = jnp.clip(off + base, 0.0, bound)

    lio = lax.broadcasted_iota(jnp.int32, (Cout // (s * s), sWs), 1)
    idx = ((lio // (s * W)) * (s * W) + (lio % s) * W
           + (lio % (s * W)) // s)
    for h in range(H):
        parts = [pos[k * 8:(k + 1) * 8, h * W:(h + 1) * W]
                 for k in range(s * s)]
        cat = jnp.concatenate(parts, axis=1)
        o_ref[0, :, h, :] = jnp.take_along_axis(cat, idx, axis=1)


def _position_head(x2, wp, bp, *, H, W, s, G):
    B, C, N = x2.shape
    Cout = wp.shape[0]
    kern = functools.partial(_pos_head_kernel, H=H, W=W, s=s)
    return pl.pallas_call(
        kern,
        out_shape=jax.ShapeDtypeStruct((B, 2 * G, H, s * W * s), jnp.float32),
        grid=(B,),
        in_specs=[
            pl.BlockSpec((1, C, N), lambda b: (b, 0, 0)),
            pl.BlockSpec((Cout, C), lambda b: (0, 0)),
            pl.BlockSpec((Cout, 1), lambda b: (0, 0)),
        ],
        out_specs=pl.BlockSpec((1, 2 * G, H, s * W * s), lambda b: (b, 0, 0, 0)),
        compiler_params=pltpu.CompilerParams(
            dimension_semantics=("parallel",)),
    )(x2, wp, bp)


def _sample_kernel(pxy_ref, x_ref, o_ref, *, H, W):
    tS = o_ref.shape[2]
    px = pxy_ref[0, 0:1, :]
    py = pxy_ref[0, 1:2, :]

    x0f = jnp.floor(px)
    y0f = jnp.floor(py)
    wx = px - x0f
    wy = py - y0f
    x0 = x0f.astype(jnp.int32)
    y0 = y0f.astype(jnp.int32)
    x1 = jnp.minimum(x0 + 1, W - 1)
    y1 = jnp.minimum(y0 + 1, H - 1)

    wio = lax.broadcasted_iota(jnp.int32, (W, tS), 0)
    colT = (jnp.where(wio == x0, 1.0 - wx, 0.0)
            + jnp.where(wio == x1, wx, 0.0)).astype(jnp.bfloat16)
    hio = lax.broadcasted_iota(jnp.int32, (H, tS), 0)
    rowT = (jnp.where(hio == y0, 1.0 - wy, 0.0)
            + jnp.where(hio == y1, wy, 0.0)).astype(jnp.bfloat16)

    m = (rowT[:, None, :] * colT[None, :, :]).reshape(H * W, tS)
    xb = x_ref[0].astype(jnp.bfloat16)
    o_ref[0] = jnp.dot(xb, m, preferred_element_type=jnp.float32)


def _bilinear_sample(pxy, x_pix, *, H, W, tS):
    BG, Cg, N = x_pix.shape
    S = pxy.shape[2]
    kern = functools.partial(_sample_kernel, H=H, W=W)
    return pl.pallas_call(
        kern,
        out_shape=jax.ShapeDtypeStruct((BG, Cg, S), jnp.float32),
        grid=(BG, S // tS),
        in_specs=[
            pl.BlockSpec((1, 2, tS), lambda i, j: (i, 0, j)),
            pl.BlockSpec((1, Cg, N), lambda i, j: (i, 0, 0)),
        ],
        out_specs=pl.BlockSpec((1, Cg, tS), lambda i, j: (i, 0, j)),
        compiler_params=pltpu.CompilerParams(
            dimension_semantics=("parallel", "parallel")),
    )(pxy, x_pix)


def _init_pos(scale, groups):
    h = (jnp.arange(scale, dtype=jnp.float32) - (scale - 1) / 2.0) / scale
    t0 = jnp.broadcast_to(h[None, :], (scale, scale))
    t1 = jnp.broadcast_to(h[:, None], (scale, scale))
    return jnp.tile(jnp.stack([t0, t1]), (1, groups, 1)).reshape(-1)


def kernel(x, weight, bias):
    B, C, H, W = x.shape
    Cout = weight.shape[0]
    G, s = 4, 2
    Cg = C // G
    S = s * H * W * s
    N = H * W

    perm = jnp.array([((ax * G + g) * s + sy) * s + sx
                      for sy in range(s) for sx in range(s)
                      for g in range(G) for ax in range(2)], dtype=jnp.int32)
    wp = (weight * 0.25).astype(jnp.float32)[perm]
    bp = (bias * 0.25 + _init_pos(s, G)).astype(jnp.float32)[perm]
    bp = bp.reshape(Cout, 1)

    pos = _position_head(x.reshape(B, C, N), wp, bp, H=H, W=W, s=s, G=G)
    pxy = pos.reshape(B * G, 2, S)

    x_pix = x.reshape(B * G, Cg, N)

    tS = 2048 if S % 2048 == 0 else S
    samp = _bilinear_sample(pxy, x_pix, H=H, W=W, tS=tS)
    return samp.reshape(B, C, s * H, s * W)


# speedup vs baseline: 3.2980x; 1.0091x over previous
import functools

import jax
import jax.numpy as jnp
from jax import lax
from jax.experimental import pallas as pl
from jax.experimental.pallas import tpu as pltpu


def _pos_head_kernel(x_ref, w_ref, b_ref, px_ref, py_ref, *, H, W, s, G):
    N = x_ref.shape[2]
    Cout = w_ref.shape[0]
    sWs = s * W * s
    off = (jnp.dot(w_ref[...], x_ref[0], preferred_element_type=jnp.float32)
           + b_ref[...])
    nio = lax.broadcasted_iota(jnp.int32, (1, N), 1)
    ch = lax.broadcasted_iota(jnp.int32, (Cout, 1), 0)
    is_x = ((ch // G) % 2) == 0
    base = jnp.where(is_x, nio % W, nio // W).astype(jnp.float32)
    bound = jnp.where(is_x, float(W - 1), float(H - 1))
    pos = jnp.clip(off + base, 0.0, bound)

    lio = lax.broadcasted_iota(jnp.int32, (2 * G, sWs), 1)
    idx = ((lio // (s * W)) * (s * W) + (lio % s) * W
           + (lio % (s * W)) // s)
    for h in range(H):
        parts = [pos[k * 2 * G:(k + 1) * 2 * G, h * W:(h + 1) * W]
                 for k in range(s * s)]
        cat = jnp.concatenate(parts, axis=1)
        frag = jnp.take_along_axis(cat, idx, axis=1)
        px_ref[0, :, h, :] = frag[0:G]
        py_ref[0, :, h, :] = frag[G:2 * G]


def _position_head(x2, wp, bp, *, H, W, s, G):
    B, C, N = x2.shape
    Cout = wp.shape[0]
    kern = functools.partial(_pos_head_kernel, H=H, W=W, s=s, G=G)
    oshape = jax.ShapeDtypeStruct((B, G, H, s * W * s), jnp.float32)
    return pl.pallas_call(
        kern,
        out_shape=(oshape, oshape),
        grid=(B,),
        in_specs=[
            pl.BlockSpec((1, C, N), lambda b: (b, 0, 0)),
            pl.BlockSpec((Cout, C), lambda b: (0, 0)),
            pl.BlockSpec((Cout, 1), lambda b: (0, 0)),
        ],
        out_specs=(
            pl.BlockSpec((1, G, H, s * W * s), lambda b: (b, 0, 0, 0)),
            pl.BlockSpec((1, G, H, s * W * s), lambda b: (b, 0, 0, 0)),
        ),
        compiler_params=pltpu.CompilerParams(
            dimension_semantics=("parallel",)),
    )(x2, wp, bp)


def _sample_kernel(px_ref, py_ref, x_ref, o_ref, *, H, W):
    tS = o_ref.shape[2]
    px = px_ref[0]
    py = py_ref[0]

    x0f = jnp.floor(px)
    y0f = jnp.floor(py)
    wx = px - x0f
    wy = py - y0f
    x0 = x0f.astype(jnp.int32)
    y0 = y0f.astype(jnp.int32)
    x1 = jnp.minimum(x0 + 1, W - 1)
    y1 = jnp.minimum(y0 + 1, H - 1)

    wio = lax.broadcasted_iota(jnp.int32, (W, tS), 0)
    colT = (jnp.where(wio == x0, 1.0 - wx, 0.0)
            + jnp.where(wio == x1, wx, 0.0)).astype(jnp.bfloat16)
    hio = lax.broadcasted_iota(jnp.int32, (H, tS), 0)
    rowT = (jnp.where(hio == y0, 1.0 - wy, 0.0)
            + jnp.where(hio == y1, wy, 0.0)).astype(jnp.bfloat16)

    m = (rowT[:, None, :] * colT[None, :, :]).reshape(H * W, tS)
    xb = x_ref[0].astype(jnp.bfloat16)
    o_ref[0] = jnp.dot(xb, m, preferred_element_type=jnp.float32)


def _bilinear_sample(px, py, x_pix, *, H, W, tS):
    BG, Cg, N = x_pix.shape
    S = px.shape[2]
    kern = functools.partial(_sample_kernel, H=H, W=W)
    return pl.pallas_call(
        kern,
        out_shape=jax.ShapeDtypeStruct((BG, Cg, S), jnp.float32),
        grid=(BG, S // tS),
        in_specs=[
            pl.BlockSpec((1, 1, tS), lambda i, j: (i, 0, j)),
            pl.BlockSpec((1, 1, tS), lambda i, j: (i, 0, j)),
            pl.BlockSpec((1, Cg, N), lambda i, j: (i, 0, 0)),
        ],
        out_specs=pl.BlockSpec((1, Cg, tS), lambda i, j: (i, 0, j)),
        compiler_params=pltpu.CompilerParams(
            dimension_semantics=("parallel", "arbitrary")),
    )(px, py, x_pix)


def _init_pos(scale, groups):
    h = (jnp.arange(scale, dtype=jnp.float32) - (scale - 1) / 2.0) / scale
    t0 = jnp.broadcast_to(h[None, :], (scale, scale))
    t1 = jnp.broadcast_to(h[:, None], (scale, scale))
    return jnp.tile(jnp.stack([t0, t1]), (1, groups, 1)).reshape(-1)


def kernel(x, weight, bias):
    B, C, H, W = x.shape
    Cout = weight.shape[0]
    G, s = 4, 2
    Cg = C // G
    S = s * H * W * s
    N = H * W

    perm = jnp.array([((ax * G + g) * s + sy) * s + sx
                      for sy in range(s) for sx in range(s)
                      for ax in range(2) for g in range(G)], dtype=jnp.int32)
    wp = (weight * 0.25).astype(jnp.float32)[perm]
    bp = (bias * 0.25 + _init_pos(s, G)).astype(jnp.float32)[perm]
    bp = bp.reshape(Cout, 1)

    pxo, pyo = _position_head(x.reshape(B, C, N), wp, bp, H=H, W=W, s=s, G=G)
    px = pxo.reshape(B * G, 1, S)
    py = pyo.reshape(B * G, 1, S)

    x_pix = x.reshape(B * G, Cg, N)

    tS = 2048 if S % 2048 == 0 else S
    samp = _bilinear_sample(px, py, x_pix, H=H, W=W, tS=tS)
    return samp.reshape(B, C, s * H, s * W)


# speedup vs baseline: 3.7442x; 1.1353x over previous
import functools

import jax
import jax.numpy as jnp
from jax import lax
from jax.experimental import pallas as pl
from jax.experimental.pallas import tpu as pltpu


def _pos_head_kernel(x_ref, w_ref, b_ref, px_ref, py_ref, *, H, W, s, G):
    N = x_ref.shape[2]
    Cout = w_ref.shape[0]
    sWs = s * W * s
    off = (jnp.dot(w_ref[...], x_ref[0], preferred_element_type=jnp.float32)
           + b_ref[...])
    nio = lax.broadcasted_iota(jnp.int32, (1, N), 1)
    ch = lax.broadcasted_iota(jnp.int32, (Cout, 1), 0)
    is_x = ((ch // G) % 2) == 0
    base = jnp.where(is_x, nio % W, nio // W).astype(jnp.float32)
    bound = jnp.where(is_x, float(W - 1), float(H - 1))
    pos = jnp.clip(off + base, 0.0, bound)

    lio = lax.broadcasted_iota(jnp.int32, (2 * G, sWs), 1)
    idx = ((lio // (s * W)) * (s * W) + (lio % s) * W
           + (lio % (s * W)) // s)
    for h in range(H):
        parts = [pos[k * 2 * G:(k + 1) * 2 * G, h * W:(h + 1) * W]
                 for k in range(s * s)]
        cat = jnp.concatenate(parts, axis=1)
        frag = jnp.take_along_axis(cat, idx, axis=1)
        px_ref[0, :, h, :] = frag[0:G]
        py_ref[0, :, h, :] = frag[G:2 * G]


def _position_head(x2, wp, bp, *, H, W, s, G):
    B, C, N = x2.shape
    Cout = wp.shape[0]
    kern = functools.partial(_pos_head_kernel, H=H, W=W, s=s, G=G)
    oshape = jax.ShapeDtypeStruct((B, G, H, s * W * s), jnp.float32)
    return pl.pallas_call(
        kern,
        out_shape=(oshape, oshape),
        grid=(B,),
        in_specs=[
            pl.BlockSpec((1, C, N), lambda b: (b, 0, 0)),
            pl.BlockSpec((Cout, C), lambda b: (0, 0)),
            pl.BlockSpec((Cout, 1), lambda b: (0, 0)),
        ],
        out_specs=(
            pl.BlockSpec((1, G, H, s * W * s), lambda b: (b, 0, 0, 0)),
            pl.BlockSpec((1, G, H, s * W * s), lambda b: (b, 0, 0, 0)),
        ),
        compiler_params=pltpu.CompilerParams(
            dimension_semantics=("parallel",)),
    )(x2, wp, bp)


def _sample_kernel(px_ref, py_ref, x_ref, o_ref, *, H, W):
    tS = o_ref.shape[2]
    px = px_ref[0]
    py = py_ref[0]

    x0f = jnp.floor(px)
    y0f = jnp.floor(py)
    wx = px - x0f
    wy = py - y0f
    x0 = x0f.astype(jnp.int32)
    y0 = y0f.astype(jnp.int32)
    x1 = jnp.minimum(x0 + 1, W - 1)
    y1 = jnp.minimum(y0 + 1, H - 1)

    wio = lax.broadcasted_iota(jnp.int32, (W, tS), 0)
    colT = (jnp.where(wio == x0, 1.0 - wx, 0.0)
            + jnp.where(wio == x1, wx, 0.0)).astype(jnp.bfloat16)
    hio = lax.broadcasted_iota(jnp.int32, (H, tS), 0)
    rowT = (jnp.where(hio == y0, 1.0 - wy, 0.0)
            + jnp.where(hio == y1, wy, 0.0)).astype(jnp.bfloat16)

    m = (rowT[:, None, :] * colT[None, :, :]).reshape(H * W, tS)
    xb = x_ref[0].astype(jnp.bfloat16)
    o_ref[0] = jnp.dot(xb, m, preferred_element_type=jnp.float32)


def _bilinear_sample(px, py, x2, *, H, W, G, tS):
    B, C, N = x2.shape
    Cg = C // G
    BG = B * G
    S = px.shape[2]
    kern = functools.partial(_sample_kernel, H=H, W=W)
    return pl.pallas_call(
        kern,
        out_shape=jax.ShapeDtypeStruct((BG, Cg, S), jnp.float32),
        grid=(BG, S // tS),
        in_specs=[
            pl.BlockSpec((1, 1, tS), lambda i, j: (i, 0, j)),
            pl.BlockSpec((1, 1, tS), lambda i, j: (i, 0, j)),
            pl.BlockSpec((1, Cg, N), lambda i, j: (i // G, i % G, 0)),
        ],
        out_specs=pl.BlockSpec((1, Cg, tS), lambda i, j: (i, 0, j)),
        compiler_params=pltpu.CompilerParams(
            dimension_semantics=("parallel", "arbitrary")),
    )(px, py, x2)


def _init_pos(scale, groups):
    h = (jnp.arange(scale, dtype=jnp.float32) - (scale - 1) / 2.0) / scale
    t0 = jnp.broadcast_to(h[None, :], (scale, scale))
    t1 = jnp.broadcast_to(h[:, None], (scale, scale))
    return jnp.tile(jnp.stack([t0, t1]), (1, groups, 1)).reshape(-1)


def kernel(x, weight, bias):
    B, C, H, W = x.shape
    Cout = weight.shape[0]
    G, s = 4, 2
    Cg = C // G
    S = s * H * W * s
    N = H * W

    perm = jnp.array([((ax * G + g) * s + sy) * s + sx
                      for sy in range(s) for sx in range(s)
                      for ax in range(2) for g in range(G)], dtype=jnp.int32)
    wp = (weight * 0.25).astype(jnp.float32)[perm]
    bp = (bias * 0.25 + _init_pos(s, G)).astype(jnp.float32)[perm]
    bp = bp.reshape(Cout, 1)

    x2 = x.reshape(B, C, N)
    pxo, pyo = _position_head(x2, wp, bp, H=H, W=W, s=s, G=G)
    px = pxo.reshape(B * G, 1, S)
    py = pyo.reshape(B * G, 1, S)

    tS = 2048 if S % 2048 == 0 else S
    samp = _bilinear_sample(px, py, x2, H=H, W=W, G=G, tS=tS)
    return samp.reshape(B, C, s * H, s * W)


# speedup vs baseline: 4.0345x; 1.0775x over previous
import functools

import jax
import jax.numpy as jnp
from jax import lax
from jax.experimental import pallas as pl
from jax.experimental.pallas import tpu as pltpu


def _pos_head_kernel(x_ref, w_ref, b_ref, px_ref, py_ref, *, H, W, s, G):
    N = x_ref.shape[2]
    Cout = w_ref.shape[0]
    sWs = s * W * s
    off = (jnp.dot(w_ref[...], x_ref[0], preferred_element_type=jnp.float32)
           + b_ref[...])
    nio = lax.broadcasted_iota(jnp.int32, (1, N), 1)
    ch = lax.broadcasted_iota(jnp.int32, (Cout, 1), 0)
    is_x = ((ch // G) % 2) == 0
    base = jnp.where(is_x, nio % W, nio // W).astype(jnp.float32)
    bound = jnp.where(is_x, float(W - 1), float(H - 1))
    pos = jnp.clip(off + base, 0.0, bound)

    lio = lax.broadcasted_iota(jnp.int32, (2 * G, sWs), 1)
    idx = ((lio // (s * W)) * (s * W) + (lio % s) * W
           + (lio % (s * W)) // s)
    for h in range(H):
        parts = [pos[k * 2 * G:(k + 1) * 2 * G, h * W:(h + 1) * W]
                 for k in range(s * s)]
        cat = jnp.concatenate(parts, axis=1)
        frag = jnp.take_along_axis(cat, idx, axis=1)
        px_ref[0, :, h, :] = frag[0:G]
        py_ref[0, :, h, :] = frag[G:2 * G]


def _position_head(x2, wp, bp, *, H, W, s, G):
    B, C, N = x2.shape
    Cout = wp.shape[0]
    kern = functools.partial(_pos_head_kernel, H=H, W=W, s=s, G=G)
    oshape = jax.ShapeDtypeStruct((B, G, H, s * W * s), jnp.float32)
    return pl.pallas_call(
        kern,
        out_shape=(oshape, oshape),
        grid=(B,),
        in_specs=[
            pl.BlockSpec((1, C, N), lambda b: (b, 0, 0)),
            pl.BlockSpec((Cout, C), lambda b: (0, 0)),
            pl.BlockSpec((Cout, 1), lambda b: (0, 0)),
        ],
        out_specs=(
            pl.BlockSpec((1, G, H, s * W * s), lambda b: (b, 0, 0, 0)),
            pl.BlockSpec((1, G, H, s * W * s), lambda b: (b, 0, 0, 0)),
        ),
        compiler_params=pltpu.CompilerParams(
            dimension_semantics=("parallel",)),
    )(x2, wp, bp)


def _sample_kernel(px_ref, py_ref, x_ref, o_ref, *, H, W):
    tS = o_ref.shape[2]
    px = px_ref[0]
    py = py_ref[0]

    x0f = jnp.floor(px)
    y0f = jnp.floor(py)
    wx = px - x0f
    wy = py - y0f
    x0 = x0f.astype(jnp.int32)
    y0 = y0f.astype(jnp.int32)
    x1 = jnp.minimum(x0 + 1, W - 1)
    y1 = jnp.minimum(y0 + 1, H - 1)

    wio = lax.broadcasted_iota(jnp.int32, (W, tS), 0)
    colT = (jnp.where(wio == x0, 1.0 - wx, 0.0)
            + jnp.where(wio == x1, wx, 0.0)).astype(jnp.bfloat16)
    hio = lax.broadcasted_iota(jnp.int32, (H, tS), 0)
    rowT = (jnp.where(hio == y0, 1.0 - wy, 0.0)
            + jnp.where(hio == y1, wy, 0.0)).astype(jnp.bfloat16)

    m = (rowT[:, None, :] * colT[None, :, :]).reshape(H * W, tS)
    xb = x_ref[0].astype(jnp.bfloat16)
    o_ref[0] = jnp.dot(xb, m, preferred_element_type=jnp.float32)


def _bilinear_sample(px, py, x2, *, H, W, G, tS):
    B, C, N = x2.shape
    Cg = C // G
    BG = B * G
    S = px.shape[2]
    kern = functools.partial(_sample_kernel, H=H, W=W)
    return pl.pallas_call(
        kern,
        out_shape=jax.ShapeDtypeStruct((BG, Cg, S), jnp.float32),
        grid=(BG, S // tS),
        in_specs=[
            pl.BlockSpec((1, 1, tS), lambda i, j: (i, 0, j)),
            pl.BlockSpec((1, 1, tS), lambda i, j: (i, 0, j)),
            pl.BlockSpec((1, Cg, N), lambda i, j: (i // G, i % G, 0)),
        ],
        out_specs=pl.BlockSpec((1, Cg, tS), lambda i, j: (i, 0, j)),
        compiler_params=pltpu.CompilerParams(
            dimension_semantics=("parallel", "arbitrary")),
    )(px, py, x2)


def _init_pos(scale, groups):
    h = (jnp.arange(scale, dtype=jnp.float32) - (scale - 1) / 2.0) / scale
    t0 = jnp.broadcast_to(h[None, :], (scale, scale))
    t1 = jnp.broadcast_to(h[:, None], (scale, scale))
    return jnp.tile(jnp.stack([t0, t1]), (1, groups, 1)).reshape(-1)


def kernel(x, weight, bias):
    B, C, H, W = x.shape
    Cout = weight.shape[0]
    G, s = 4, 2
    Cg = C // G
    S = s * H * W * s
    N = H * W

    perm = jnp.array([((ax * G + g) * s + sy) * s + sx
                      for sy in range(s) for sx in range(s)
                      for ax in range(2) for g in range(G)], dtype=jnp.int32)
    wp = (weight * 0.25).astype(jnp.float32)[perm]
    bp = (bias * 0.25 + _init_pos(s, G)).astype(jnp.float32)[perm]
    bp = bp.reshape(Cout, 1)

    x2 = x.reshape(B, C, N)
    pxo, pyo = _position_head(x2, wp, bp, H=H, W=W, s=s, G=G)
    px = pxo.reshape(B * G, 1, S)
    py = pyo.reshape(B * G, 1, S)

    tS = 4096 if S % 4096 == 0 else S
    samp = _bilinear_sample(px, py, x2, H=H, W=W, G=G, tS=tS)
    return samp.reshape(B, C, s * H, s * W)


# speedup vs baseline: 5.6305x; 1.3956x over previous
import functools

import jax
import jax.numpy as jnp
from jax import lax
from jax.experimental import pallas as pl
from jax.experimental.pallas import tpu as pltpu


def _dysample_kernel(x_ref, w_ref, b_ref, o_ref, *, H, W, s, G):
    N = x_ref.shape[2]
    Cout = w_ref.shape[0]
    C = x_ref.shape[1]
    Cg = C // G
    sWs = s * W * s

    off = (jnp.dot(w_ref[...], x_ref[0], preferred_element_type=jnp.float32)
           + b_ref[...])
    nio = lax.broadcasted_iota(jnp.int32, (1, N), 1)
    ch = lax.broadcasted_iota(jnp.int32, (Cout, 1), 0)
    is_x = ((ch // G) % 2) == 0
    base = jnp.where(is_x, nio % W, nio // W).astype(jnp.float32)
    bound = jnp.where(is_x, float(W - 1), float(H - 1))
    pos = jnp.clip(off + base, 0.0, bound)

    lio = lax.broadcasted_iota(jnp.int32, (2 * G, sWs), 1)
    idx = ((lio // (s * W)) * (s * W) + (lio % s) * W
           + (lio % (s * W)) // s)
    frags = []
    for h in range(H):
        parts = [pos[k * 2 * G:(k + 1) * 2 * G, h * W:(h + 1) * W]
                 for k in range(s * s)]
        cat = jnp.concatenate(parts, axis=1)
        frags.append(jnp.take_along_axis(cat, idx, axis=1))
    pxy = jnp.concatenate(frags, axis=1)

    xb = x_ref[0].astype(jnp.bfloat16)
    wio = lax.broadcasted_iota(jnp.int32, (W, H * sWs), 0)
    hio = lax.broadcasted_iota(jnp.int32, (H, H * sWs), 0)
    for g in range(G):
        px = pxy[g:g + 1, :]
        py = pxy[G + g:G + g + 1, :]
        x0f = jnp.floor(px)
        y0f = jnp.floor(py)
        wx = px - x0f
        wy = py - y0f
        x0 = x0f.astype(jnp.int32)
        y0 = y0f.astype(jnp.int32)
        x1 = jnp.minimum(x0 + 1, W - 1)
        y1 = jnp.minimum(y0 + 1, H - 1)
        colT = (jnp.where(wio == x0, 1.0 - wx, 0.0)
                + jnp.where(wio == x1, wx, 0.0)).astype(jnp.bfloat16)
        rowT = (jnp.where(hio == y0, 1.0 - wy, 0.0)
                + jnp.where(hio == y1, wy, 0.0)).astype(jnp.bfloat16)
        m = (rowT[:, None, :] * colT[None, :, :]).reshape(H * W, H * sWs)
        o_ref[0, g * Cg:(g + 1) * Cg, :] = jnp.dot(
            xb[g * Cg:(g + 1) * Cg, :], m, preferred_element_type=jnp.float32)


def _dysample(x2, wp, bp, *, H, W, s, G):
    B, C, N = x2.shape
    Cout = wp.shape[0]
    S = s * s * N
    kern = functools.partial(_dysample_kernel, H=H, W=W, s=s, G=G)
    return pl.pallas_call(
        kern,
        out_shape=jax.ShapeDtypeStruct((B, C, S), jnp.float32),
        grid=(B,),
        in_specs=[
            pl.BlockSpec((1, C, N), lambda b: (b, 0, 0)),
            pl.BlockSpec((Cout, C), lambda b: (0, 0)),
            pl.BlockSpec((Cout, 1), lambda b: (0, 0)),
        ],
        out_specs=pl.BlockSpec((1, C, S), lambda b: (b, 0, 0)),
        compiler_params=pltpu.CompilerParams(
            dimension_semantics=("parallel",)),
    )(x2, wp, bp)


def _init_pos(scale, groups):
    h = (jnp.arange(scale, dtype=jnp.float32) - (scale - 1) / 2.0) / scale
    t0 = jnp.broadcast_to(h[None, :], (scale, scale))
    t1 = jnp.broadcast_to(h[:, None], (scale, scale))
    return jnp.tile(jnp.stack([t0, t1]), (1, groups, 1)).reshape(-1)


def kernel(x, weight, bias):
    B, C, H, W = x.shape
    Cout = weight.shape[0]
    G, s = 4, 2
    N = H * W

    perm = jnp.array([((ax * G + g) * s + sy) * s + sx
                      for sy in range(s) for sx in range(s)
                      for ax in range(2) for g in range(G)], dtype=jnp.int32)
    wp = (weight * 0.25).astype(jnp.float32)[perm]
    bp = (bias * 0.25 + _init_pos(s, G)).astype(jnp.float32)[perm]
    bp = bp.reshape(Cout, 1)

    x2 = x.reshape(B, C, N)
    samp = _dysample(x2, wp, bp, H=H, W=W, s=s, G=G)
    return samp.reshape(B, C, s * H, s * W)


# speedup vs baseline: 5.6326x; 1.0004x over previous
import functools

import jax
import jax.numpy as jnp
from jax import lax
from jax.experimental import pallas as pl
from jax.experimental.pallas import tpu as pltpu


def _dysample_kernel(x_ref, w_ref, b_ref, o_ref, *, H, W, s, G):
    N = x_ref.shape[2]
    Cout = w_ref.shape[0]
    C = x_ref.shape[1]
    Cg = C // G
    sWs = s * W * s

    off = (jnp.dot(w_ref[...], x_ref[0], preferred_element_type=jnp.float32)
           + b_ref[...])
    nio = lax.broadcasted_iota(jnp.int32, (1, N), 1)
    ch = lax.broadcasted_iota(jnp.int32, (Cout, 1), 0)
    is_x = ((ch // G) % 2) == 0
    base = jnp.where(is_x, nio % W, nio // W).astype(jnp.float32)
    bound = jnp.where(is_x, float(W - 1), float(H - 1))
    pos = jnp.clip(off + base, 0.0, bound)

    lio = lax.broadcasted_iota(jnp.int32, (2 * G, sWs), 1)
    idx = ((lio // (s * W)) * (s * W) + (lio % s) * W
           + (lio % (s * W)) // s)
    frags = []
    for h in range(H):
        parts = [pos[k * 2 * G:(k + 1) * 2 * G, h * W:(h + 1) * W]
                 for k in range(s * s)]
        cat = jnp.concatenate(parts, axis=1)
        frags.append(jnp.take_along_axis(cat, idx, axis=1))
    pxy = jnp.concatenate(frags, axis=1)

    xb = x_ref[0].astype(jnp.bfloat16)
    S = H * sWs
    nchunk = 2 if S % 2 == 0 else 1
    tS = S // nchunk
    wio = lax.broadcasted_iota(jnp.int32, (W, tS), 0)
    hio = lax.broadcasted_iota(jnp.int32, (H, tS), 0)
    for g in range(G):
        for c in range(nchunk):
            sl = slice(c * tS, (c + 1) * tS)
            px = pxy[g:g + 1, sl]
            py = pxy[G + g:G + g + 1, sl]
            x0f = jnp.floor(px)
            y0f = jnp.floor(py)
            wx = px - x0f
            wy = py - y0f
            x0 = x0f.astype(jnp.int32)
            y0 = y0f.astype(jnp.int32)
            x1 = jnp.minimum(x0 + 1, W - 1)
            y1 = jnp.minimum(y0 + 1, H - 1)
            colT = (jnp.where(wio == x0, 1.0 - wx, 0.0)
                    + jnp.where(wio == x1, wx, 0.0)).astype(jnp.bfloat16)
            rowT = (jnp.where(hio == y0, 1.0 - wy, 0.0)
                    + jnp.where(hio == y1, wy, 0.0)).astype(jnp.bfloat16)
            m = (rowT[:, None, :] * colT[None, :, :]).reshape(H * W, tS)
            o_ref[0, g * Cg:(g + 1) * Cg, sl] = jnp.dot(
                xb[g * Cg:(g + 1) * Cg, :], m,
                preferred_element_type=jnp.float32)


def _dysample(x2, wp, bp, *, H, W, s, G):
    B, C, N = x2.shape
    Cout = wp.shape[0]
    S = s * s * N
    kern = functools.partial(_dysample_kernel, H=H, W=W, s=s, G=G)
    return pl.pallas_call(
        kern,
        out_shape=jax.ShapeDtypeStruct((B, C, S), jnp.float32),
        grid=(B,),
        in_specs=[
            pl.BlockSpec((1, C, N), lambda b: (b, 0, 0)),
            pl.BlockSpec((Cout, C), lambda b: (0, 0)),
            pl.BlockSpec((Cout, 1), lambda b: (0, 0)),
        ],
        out_specs=pl.BlockSpec((1, C, S), lambda b: (b, 0, 0)),
        compiler_params=pltpu.CompilerParams(
            dimension_semantics=("parallel",)),
    )(x2, wp, bp)


def _init_pos(scale, groups):
    h = (jnp.arange(scale, dtype=jnp.float32) - (scale - 1) / 2.0) / scale
    t0 = jnp.broadcast_to(h[None, :], (scale, scale))
    t1 = jnp.broadcast_to(h[:, None], (scale, scale))
    return jnp.tile(jnp.stack([t0, t1]), (1, groups, 1)).reshape(-1)


def kernel(x, weight, bias):
    B, C, H, W = x.shape
    Cout = weight.shape[0]
    G, s = 4, 2
    N = H * W

    perm = jnp.array([((ax * G + g) * s + sy) * s + sx
                      for sy in range(s) for sx in range(s)
                      for ax in range(2) for g in range(G)], dtype=jnp.int32)
    wp = (weight * 0.25).astype(jnp.float32)[perm]
    bp = (bias * 0.25 + _init_pos(s, G)).astype(jnp.float32)[perm]
    bp = bp.reshape(Cout, 1)

    x2 = x.reshape(B, C, N)
    samp = _dysample(x2, wp, bp, H=H, W=W, s=s, G=G)
    return samp.reshape(B, C, s * H, s * W)
